# Initial kernel scaffold; baseline (speedup 1.0000x reference)
import jax
import jax.numpy as jnp
from jax.experimental import pallas as pl


def kernel(radar_tensor):
    raise NotImplementedError("write your pallas kernel here")


# speedup vs baseline: 1.0241x; 1.0241x over previous
import functools

import jax
import jax.numpy as jnp
from jax.experimental import pallas as pl
from jax.experimental.pallas import tpu as pltpu

K = 100
B, D, R, E, A = 2, 32, 256, 37, 107
N = E * A
R_BLK = 16


def _mean_body(x_ref, o_ref):
    o_ref[...] = jnp.sum(x_ref[...], axis=1) * (1.0 / D)


def _mean_cube(radar_tensor):
    grid = (B, R // R_BLK)
    return pl.pallas_call(
        _mean_body,
        grid=grid,
        in_specs=[pl.BlockSpec((1, D, R_BLK, E, A), lambda b, r: (b, 0, r, 0, 0))],
        out_specs=pl.BlockSpec((1, R_BLK, E, A), lambda b, r: (b, r, 0, 0)),
        out_shape=jax.ShapeDtypeStruct((B, R, E, A), jnp.float32),
    )(radar_tensor)


def kernel(radar_tensor):
    cube = _mean_cube(radar_tensor)
    cube_flat = cube.reshape(B, R, N)
    _, top_k_idx = jax.lax.top_k(cube_flat, K)
    power_val = jnp.take_along_axis(cube_flat, top_k_idx, axis=2)
    elevation_ind = top_k_idx // A
    azimuth_ind = top_k_idx % A
    range_ind = jnp.broadcast_to(jnp.arange(R)[None, :, None], (B, R, K))
    sparse_rdr_cube = jnp.stack(
        [
            azimuth_ind.astype(jnp.float32),
            range_ind.astype(jnp.float32),
            elevation_ind.astype(jnp.float32),
            power_val,
        ],
        axis=-1,
    ).reshape(B * R * K, 4)
    batch_ind = jnp.broadcast_to(jnp.arange(B)[:, None, None], (B, R, K))
    sp_indices = (
        jnp.stack([batch_ind, elevation_ind + 1, range_ind, azimuth_ind + 74], axis=-1)
        .reshape(B * R * K, 4)
        .astype(jnp.int32)
    )
    return sparse_rdr_cube, sp_indices
